# Initial kernel scaffold; baseline (speedup 1.0000x reference)
import jax
import jax.numpy as jnp
from jax.experimental import pallas as pl


def kernel(x, edge_index, W_atten, W1, W2, gamma, beta):
    raise NotImplementedError("write your pallas kernel here")


# speedup vs baseline: 6.4498x; 6.4498x over previous
import functools

import jax
import jax.numpy as jnp
from jax import lax
from jax.experimental import pallas as pl
from jax.experimental.pallas import tpu as pltpu
from jax.experimental.pallas import tpu_sc as plsc

N_NODES = 10000
E = 320000
D = 128
ROW = 144
NC = 2
NS = 16
L = 16
NW = NC * NS
EDGES_PER_W = E // NW
CHUNK = 80
GROUPS = CHUNK // L
N_CHUNKS = EDGES_PER_W // CHUNK
ROWS_PER_TILE = N_NODES // NS
ZROWS = 125


def _edge_kernel(x_hbm, seg_hbm, watt_hbm, acc_hbm,
                 xbuf, ybuf, segbuf, zbuf, wbuf, shared):
    cid = lax.axis_index("c")
    sid = lax.axis_index("s")
    wid = sid * NC + cid

    zv = jnp.zeros((L,), jnp.float32)

    def zero_row(r, _):
        for j in range(ROW // L):
            zbuf[r, pl.ds(j * L, L)] = zv
        return 0

    lax.fori_loop(0, ZROWS, zero_row, 0)
    for k in range(ROWS_PER_TILE // ZROWS):
        pltpu.sync_copy(zbuf, shared.at[pl.ds(sid * ROWS_PER_TILE + k * ZROWS, ZROWS)])
    plsc.subcore_barrier()

    pltpu.sync_copy(watt_hbm, wbuf)
    wv = [wbuf[pl.ds(j * L, L)] for j in range(D // L)]
    iota = lax.iota(jnp.int32, L)
    one = jnp.ones((L,), jnp.float32)
    zero = jnp.zeros((L,), jnp.float32)

    def chunk_body(t, _):
        ebase = wid * EDGES_PER_W + t * CHUNK
        pltpu.sync_copy(x_hbm.at[pl.ds(ebase * D, CHUNK * D)], xbuf)
        pltpu.sync_copy(seg_hbm.at[pl.ds(ebase, CHUNK)], segbuf)

        def group_body(g, _):
            goff = g * (L * D)
            seg_vec = segbuf[pl.ds(g * L, L)]
            for e in range(L):
                eoff = goff + e * D
                rows = [xbuf[pl.ds(eoff + j * L, L)] for j in range(D // L)]
                acc = rows[0] * wv[0]
                for j in range(1, D // L):
                    acc = acc + rows[j] * wv[j]
                s = jnp.sum(acc)
                s = jnp.where(s >= 0.0, s, 0.01 * s)
                ev = jnp.exp(jnp.broadcast_to(s, (L,)))
                for j in range(D // L):
                    ybuf[e, pl.ds(j * L, L)] = ev * rows[j]
                tail = jnp.where(iota == 0, ev, jnp.where(iota == 1, one, zero))
                ybuf[e, pl.ds(D, L)] = tail
            pltpu.sync_copy(ybuf, shared.at[seg_vec], add=True)
            return 0

        lax.fori_loop(0, GROUPS, group_body, 0)
        return 0

    lax.fori_loop(0, N_CHUNKS, chunk_body, 0)
    plsc.subcore_barrier()

    pltpu.sync_copy(
        shared.at[pl.ds(sid * ROWS_PER_TILE, ROWS_PER_TILE)],
        acc_hbm.at[cid, pl.ds(sid * ROWS_PER_TILE, ROWS_PER_TILE)],
    )


def _dense_kernel(acc_ref, w1_ref, w2_ref, gamma_ref, beta_ref, out_ref):
    a = acc_ref[0]
    b = acc_ref[1]
    summed = a[:, :D] + b[:, :D]
    denom = a[:, D:D + 1] + b[:, D:D + 1]
    count = a[:, D + 1:D + 2] + b[:, D + 1:D + 2]
    bond = summed / (denom + 1e-16) / jnp.maximum(count, 1.0)
    wc = lax.dot_general(w2_ref[...], w1_ref[...], (((1,), (0,)), ((), ())),
                         precision=lax.Precision.HIGHEST)
    h = lax.dot_general(bond, wc, (((1,), (1,)), ((), ())),
                        precision=lax.Precision.HIGHEST)
    mu = jnp.mean(h, axis=0, keepdims=True)
    var = jnp.mean((h - mu) * (h - mu), axis=0, keepdims=True)
    hn = (h - mu) / jnp.sqrt(var + 1e-5) * gamma_ref[...] + beta_ref[...]
    out_ref[...] = 0.5 * hn * (1.0 + lax.erf(hn * 0.7071067811865475))


def kernel(x, edge_index, W_atten, W1, W2, gamma, beta):
    x_flat = x.reshape(E * D)
    seg = edge_index[1]
    watt = W_atten.reshape(D)

    mesh = plsc.VectorSubcoreMesh(core_axis_name="c", subcore_axis_name="s",
                                  num_cores=NC, num_subcores=NS)
    edge_pass = pl.kernel(
        _edge_kernel,
        out_type=jax.ShapeDtypeStruct((NC, N_NODES, ROW), jnp.float32),
        mesh=mesh,
        compiler_params=pltpu.CompilerParams(use_tc_tiling_on_sc=False,
                                             needs_layout_passes=False),
        scratch_types=[
            pltpu.VMEM((CHUNK * D,), jnp.float32),
            pltpu.VMEM((L, ROW), jnp.float32),
            pltpu.VMEM((CHUNK,), jnp.int32),
            pltpu.VMEM((ZROWS, ROW), jnp.float32),
            pltpu.VMEM((D,), jnp.float32),
            pltpu.VMEM_SHARED((N_NODES, ROW), jnp.float32),
        ],
    )
    acc = edge_pass(x_flat, seg, watt)

    out = pl.pallas_call(
        _dense_kernel,
        out_shape=jax.ShapeDtypeStruct((N_NODES, D), jnp.float32),
    )(acc, W1, W2, gamma.reshape(1, D), beta.reshape(1, D))
    return out


# speedup vs baseline: 9.5548x; 1.4814x over previous
import functools

import jax
import jax.numpy as jnp
from jax import lax
from jax.experimental import pallas as pl
from jax.experimental.pallas import tpu as pltpu
from jax.experimental.pallas import tpu_sc as plsc

N_NODES = 10000
E = 320000
D = 128
ROW = 144
NC = 2
NS = 16
L = 16
NW = NC * NS
EDGES_PER_W = E // NW
CHUNK = 64
GROUPS = CHUNK // L
N_CHUNKS = 156
LEFT = EDGES_PER_W - N_CHUNKS * CHUNK
ROWS_PER_TILE = N_NODES // NS


def _edge_kernel(x_hbm, seg_hbm, watt_hbm, acc_hbm,
                 xA, xB, sgA, sgB, yA, yB, wbuf, shared,
                 semA, semB, semyA, semyB):
    cid = lax.axis_index("c")
    sid = lax.axis_index("s")
    wid = sid * NC + cid
    ebase0 = wid * EDGES_PER_W

    def fetch(c, xb, sgb, sem):
        eb = ebase0 + c * CHUNK
        pltpu.async_copy(x_hbm.at[pl.ds(eb * D, CHUNK * D)], xb, sem)
        pltpu.async_copy(seg_hbm.at[pl.ds(eb, CHUNK)], sgb, sem)

    def wait_fetch(xb, sgb, sem):
        pltpu.make_async_copy(x_hbm.at[pl.ds(0, CHUNK * D)], xb, sem).wait()
        pltpu.make_async_copy(seg_hbm.at[pl.ds(0, CHUNK)], sgb, sem).wait()

    pltpu.sync_copy(watt_hbm, wbuf)
    fetch(0, xA, sgA, semA)
    fetch(1, xB, sgB, semB)

    zv = jnp.zeros((L,), jnp.float32)

    def zero_row(r, _):
        for j in range(ROW // L):
            yA[r, pl.ds(j * L, L)] = zv
        return 0

    lax.fori_loop(0, CHUNK, zero_row, 0)
    rbase = sid * ROWS_PER_TILE
    for k in range(ROWS_PER_TILE // CHUNK):
        pltpu.sync_copy(yA, shared.at[pl.ds(rbase + k * CHUNK, CHUNK)])
    rem = ROWS_PER_TILE % CHUNK
    if rem:
        pltpu.sync_copy(yA.at[pl.ds(0, rem)],
                        shared.at[pl.ds(rbase + ROWS_PER_TILE - rem, rem)])
    plsc.subcore_barrier()

    wv = [wbuf[pl.ds(j * L, L)] for j in range(D // L)]
    iota = lax.iota(jnp.int32, L)
    one = jnp.ones((L,), jnp.float32)
    zero = jnp.zeros((L,), jnp.float32)
    dummy_idx = jnp.zeros((L,), jnp.int32)

    def compute_and_scatter(xb, sgb, yb, semy, ngroups):
        def group_body(g, _):
            goff = g * (L * D)
            seg_vec = sgb[pl.ds(g * L, L)]
            for e in range(L):
                eoff = goff + e * D
                rows = [xb[pl.ds(eoff + j * L, L)] for j in range(D // L)]
                acc = rows[0] * wv[0]
                for j in range(1, D // L):
                    acc = acc + rows[j] * wv[j]
                s = jnp.sum(acc)
                s = jnp.where(s >= 0.0, s, 0.01 * s)
                ev = jnp.exp(jnp.broadcast_to(s, (L,)))
                r = g * L + e
                for j in range(D // L):
                    yb[r, pl.ds(j * L, L)] = ev * rows[j]
                tail = jnp.where(iota == 0, ev, jnp.where(iota == 1, one, zero))
                yb[r, pl.ds(D, L)] = tail
            pltpu.async_copy(yb.at[pl.ds(g * L, L)], shared.at[seg_vec], semy,
                             add=True)
            return 0

        lax.fori_loop(0, ngroups, group_body, 0)

    def wait_scatters(yb, semy, ngroups):
        for _ in range(ngroups):
            pltpu.make_async_copy(yb.at[pl.ds(0, L)], shared.at[dummy_idx],
                                  semy).wait()

    def pair_body(i, _):
        c = 2 * i
        wait_fetch(xA, sgA, semA)

        @pl.when(i > 0)
        def _():
            wait_scatters(yA, semyA, GROUPS)

        compute_and_scatter(xA, sgA, yA, semyA, GROUPS)

        @pl.when(c + 2 < N_CHUNKS)
        def _():
            fetch(c + 2, xA, sgA, semA)

        wait_fetch(xB, sgB, semB)

        @pl.when(i > 0)
        def _():
            wait_scatters(yB, semyB, GROUPS)

        compute_and_scatter(xB, sgB, yB, semyB, GROUPS)

        @pl.when(c + 3 < N_CHUNKS)
        def _():
            fetch(c + 3, xB, sgB, semB)

        return 0

    lax.fori_loop(0, N_CHUNKS // 2, pair_body, 0)

    wait_scatters(yA, semyA, GROUPS)
    lbase = ebase0 + N_CHUNKS * CHUNK
    pltpu.sync_copy(x_hbm.at[pl.ds(lbase * D, LEFT * D)], xA.at[pl.ds(0, LEFT * D)])
    pltpu.sync_copy(seg_hbm.at[pl.ds(lbase, LEFT)], sgA.at[pl.ds(0, LEFT)])
    compute_and_scatter(xA, sgA, yA, semyA, LEFT // L)
    wait_scatters(yA, semyA, LEFT // L)
    wait_scatters(yB, semyB, GROUPS)
    plsc.subcore_barrier()

    pltpu.sync_copy(
        shared.at[pl.ds(sid * ROWS_PER_TILE, ROWS_PER_TILE)],
        acc_hbm.at[cid, pl.ds(sid * ROWS_PER_TILE, ROWS_PER_TILE)],
    )


def _dense_kernel(acc_ref, w1_ref, w2_ref, gamma_ref, beta_ref, out_ref):
    a = acc_ref[0]
    b = acc_ref[1]
    summed = a[:, :D] + b[:, :D]
    denom = a[:, D:D + 1] + b[:, D:D + 1]
    count = a[:, D + 1:D + 2] + b[:, D + 1:D + 2]
    bond = summed / (denom + 1e-16) / jnp.maximum(count, 1.0)
    wc = lax.dot_general(w2_ref[...], w1_ref[...], (((1,), (0,)), ((), ())),
                         precision=lax.Precision.HIGHEST)
    h = lax.dot_general(bond, wc, (((1,), (1,)), ((), ())),
                        precision=lax.Precision.HIGHEST)
    mu = jnp.mean(h, axis=0, keepdims=True)
    var = jnp.mean((h - mu) * (h - mu), axis=0, keepdims=True)
    hn = (h - mu) / jnp.sqrt(var + 1e-5) * gamma_ref[...] + beta_ref[...]
    out_ref[...] = 0.5 * hn * (1.0 + lax.erf(hn * 0.7071067811865475))


def kernel(x, edge_index, W_atten, W1, W2, gamma, beta):
    x_flat = x.reshape(E * D)
    seg = edge_index[1]
    watt = W_atten.reshape(D)

    mesh = plsc.VectorSubcoreMesh(core_axis_name="c", subcore_axis_name="s",
                                  num_cores=NC, num_subcores=NS)
    edge_pass = pl.kernel(
        _edge_kernel,
        out_type=jax.ShapeDtypeStruct((NC, N_NODES, ROW), jnp.float32),
        mesh=mesh,
        compiler_params=pltpu.CompilerParams(use_tc_tiling_on_sc=False,
                                             needs_layout_passes=False),
        scratch_types=[
            pltpu.VMEM((CHUNK * D,), jnp.float32),
            pltpu.VMEM((CHUNK * D,), jnp.float32),
            pltpu.VMEM((CHUNK,), jnp.int32),
            pltpu.VMEM((CHUNK,), jnp.int32),
            pltpu.VMEM((CHUNK, ROW), jnp.float32),
            pltpu.VMEM((CHUNK, ROW), jnp.float32),
            pltpu.VMEM((D,), jnp.float32),
            pltpu.VMEM_SHARED((N_NODES, ROW), jnp.float32),
            pltpu.SemaphoreType.DMA,
            pltpu.SemaphoreType.DMA,
            pltpu.SemaphoreType.DMA,
            pltpu.SemaphoreType.DMA,
        ],
    )
    acc = edge_pass(x_flat, seg, watt)

    out = pl.pallas_call(
        _dense_kernel,
        out_shape=jax.ShapeDtypeStruct((N_NODES, D), jnp.float32),
    )(acc, W1, W2, gamma.reshape(1, D), beta.reshape(1, D))
    return out


# speedup vs baseline: 11.5878x; 1.2128x over previous
import functools

import jax
import jax.numpy as jnp
from jax import lax
from jax.experimental import pallas as pl
from jax.experimental.pallas import tpu as pltpu
from jax.experimental.pallas import tpu_sc as plsc

N_NODES = 10000
E = 320000
D = 128
ROW = 144
NC = 2
NS = 16
L = 16
NW = NC * NS
EDGES_PER_W = E // NW
CHUNK = 64
GROUPS = CHUNK // L
N_CHUNKS = 156
LEFT = EDGES_PER_W - N_CHUNKS * CHUNK
ROWS_PER_TILE = N_NODES // NS


def _lane_bcast(v, idx):
    return lax.gather(
        v, idx[:, None],
        lax.GatherDimensionNumbers(offset_dims=(), collapsed_slice_dims=(0,),
                                   start_index_map=(0,)),
        (1,), mode=lax.GatherScatterMode.PROMISE_IN_BOUNDS)


def _edge_kernel(x_hbm, seg_hbm, watt_hbm, acc_hbm,
                 xA, xB, sg1, sg2, sg3, sg4, yA, yB, wbuf, shared,
                 semA, semB, semyA, semyB):
    cid = lax.axis_index("c")
    sid = lax.axis_index("s")
    wid = sid * NC + cid
    ebase0 = wid * EDGES_PER_W

    def fetch(c, xb, sgb, sem):
        eb = ebase0 + c * CHUNK
        pltpu.async_copy(x_hbm.at[pl.ds(eb * D, CHUNK * D)], xb, sem)
        pltpu.async_copy(seg_hbm.at[pl.ds(eb, CHUNK)], sgb, sem)

    def wait_fetch(xb, sgb, sem):
        pltpu.make_async_copy(x_hbm.at[pl.ds(0, CHUNK * D)], xb, sem).wait()
        pltpu.make_async_copy(seg_hbm.at[pl.ds(0, CHUNK)], sgb, sem).wait()

    pltpu.sync_copy(watt_hbm, wbuf)
    fetch(0, xA, sg1, semA)
    fetch(1, xB, sg2, semB)

    zv = jnp.zeros((L,), jnp.float32)

    def zero_row(r, _):
        for j in range(ROW // L):
            yA[r, pl.ds(j * L, L)] = zv
        return 0

    lax.fori_loop(0, CHUNK, zero_row, 0)
    rbase = sid * ROWS_PER_TILE
    for k in range(ROWS_PER_TILE // CHUNK):
        pltpu.sync_copy(yA, shared.at[pl.ds(rbase + k * CHUNK, CHUNK)])
    rem = ROWS_PER_TILE % CHUNK
    if rem:
        pltpu.sync_copy(yA.at[pl.ds(0, rem)],
                        shared.at[pl.ds(rbase + ROWS_PER_TILE - rem, rem)])
    plsc.subcore_barrier()

    wv = [wbuf[pl.ds(j * L, L)] for j in range(D // L)]
    iota = lax.iota(jnp.int32, L)
    one = jnp.ones((L,), jnp.float32)
    zero = jnp.zeros((L,), jnp.float32)
    idx_last = jnp.full((L,), L - 1, jnp.int32)

    def compute(xb, yb, ngroups):
        def group_body(g, _):
            goff = g * (L * D)
            for e in range(L):
                eoff = goff + e * D
                rows = [xb[pl.ds(eoff + j * L, L)] for j in range(D // L)]
                prods = [rows[j] * wv[j] for j in range(D // L)]
                p01 = prods[0] + prods[1]
                p23 = prods[2] + prods[3]
                p45 = prods[4] + prods[5]
                p67 = prods[6] + prods[7]
                acc = (p01 + p23) + (p45 + p67)
                bc = _lane_bcast(jnp.cumsum(acc), idx_last)
                bc = jnp.where(bc >= 0.0, bc, 0.01 * bc)
                ev = jnp.exp(bc)
                r = g * L + e
                for j in range(D // L):
                    yb[r, pl.ds(j * L, L)] = ev * rows[j]
                tail = jnp.where(iota == 0, ev, jnp.where(iota == 1, one, zero))
                yb[r, pl.ds(D, L)] = tail
            return 0

        lax.fori_loop(0, ngroups, group_body, 0)

    def scatter(yb, sgb, semy):
        pltpu.async_copy(yb, shared.at[sgb], semy, add=True)

    def wait_scatter(yb, sgb, semy):
        pltpu.make_async_copy(yb, shared.at[sgb], semy).wait()

    def quad_body(i, _):
        c = 4 * i

        def phase(cc, xb, sgb, sg_next, yb, sem, semy, guard_fetch, first):
            wait_fetch(xb, sgb, sem)
            if first is not None:
                @pl.when(first)
                def _():
                    wait_scatter(yb, sgb, semy)
            else:
                wait_scatter(yb, sgb, semy)
            compute(xb, yb, GROUPS)
            scatter(yb, sgb, semy)
            if guard_fetch:
                @pl.when(cc + 2 < N_CHUNKS)
                def _():
                    fetch(cc + 2, xb, sg_next, sem)
            else:
                fetch(cc + 2, xb, sg_next, sem)

        phase(c, xA, sg1, sg3, yA, semA, semyA, False, i > 0)
        phase(c + 1, xB, sg2, sg4, yB, semB, semyB, False, i > 0)
        phase(c + 2, xA, sg3, sg1, yA, semA, semyA, True, None)
        phase(c + 3, xB, sg4, sg2, yB, semB, semyB, True, None)
        return 0

    lax.fori_loop(0, N_CHUNKS // 4, quad_body, 0)

    wait_scatter(yA, sg3, semyA)
    wait_scatter(yB, sg4, semyB)
    lbase = ebase0 + N_CHUNKS * CHUNK
    pltpu.sync_copy(x_hbm.at[pl.ds(lbase * D, LEFT * D)], xA.at[pl.ds(0, LEFT * D)])
    pltpu.sync_copy(seg_hbm.at[pl.ds(lbase, LEFT)], sg1.at[pl.ds(0, LEFT)])
    compute(xA, yA, LEFT // L)
    seg_vec = sg1[pl.ds(0, L)]
    pltpu.sync_copy(yA.at[pl.ds(0, L)], shared.at[seg_vec], add=True)
    plsc.subcore_barrier()

    pltpu.sync_copy(
        shared.at[pl.ds(sid * ROWS_PER_TILE, ROWS_PER_TILE)],
        acc_hbm.at[cid, pl.ds(sid * ROWS_PER_TILE, ROWS_PER_TILE)],
    )


def _dense_kernel(acc_ref, w1_ref, w2_ref, gamma_ref, beta_ref, out_ref):
    a = acc_ref[0]
    b = acc_ref[1]
    summed = a[:, :D] + b[:, :D]
    denom = a[:, D:D + 1] + b[:, D:D + 1]
    count = a[:, D + 1:D + 2] + b[:, D + 1:D + 2]
    bond = summed / (denom + 1e-16) / jnp.maximum(count, 1.0)
    wc = lax.dot_general(w2_ref[...], w1_ref[...], (((1,), (0,)), ((), ())),
                         precision=lax.Precision.HIGHEST)
    h = lax.dot_general(bond, wc, (((1,), (1,)), ((), ())),
                        precision=lax.Precision.HIGHEST)
    mu = jnp.mean(h, axis=0, keepdims=True)
    var = jnp.mean((h - mu) * (h - mu), axis=0, keepdims=True)
    hn = (h - mu) / jnp.sqrt(var + 1e-5) * gamma_ref[...] + beta_ref[...]
    out_ref[...] = 0.5 * hn * (1.0 + lax.erf(hn * 0.7071067811865475))


def kernel(x, edge_index, W_atten, W1, W2, gamma, beta):
    x_flat = x.reshape(E * D)
    seg = edge_index[1]
    watt = W_atten.reshape(D)

    mesh = plsc.VectorSubcoreMesh(core_axis_name="c", subcore_axis_name="s",
                                  num_cores=NC, num_subcores=NS)
    edge_pass = pl.kernel(
        _edge_kernel,
        out_type=jax.ShapeDtypeStruct((NC, N_NODES, ROW), jnp.float32),
        mesh=mesh,
        compiler_params=pltpu.CompilerParams(use_tc_tiling_on_sc=False,
                                             needs_layout_passes=False),
        scratch_types=[
            pltpu.VMEM((CHUNK * D,), jnp.float32),
            pltpu.VMEM((CHUNK * D,), jnp.float32),
            pltpu.VMEM((CHUNK,), jnp.int32),
            pltpu.VMEM((CHUNK,), jnp.int32),
            pltpu.VMEM((CHUNK,), jnp.int32),
            pltpu.VMEM((CHUNK,), jnp.int32),
            pltpu.VMEM((CHUNK, ROW), jnp.float32),
            pltpu.VMEM((CHUNK, ROW), jnp.float32),
            pltpu.VMEM((D,), jnp.float32),
            pltpu.VMEM_SHARED((N_NODES, ROW), jnp.float32),
            pltpu.SemaphoreType.DMA,
            pltpu.SemaphoreType.DMA,
            pltpu.SemaphoreType.DMA,
            pltpu.SemaphoreType.DMA,
        ],
    )
    acc = edge_pass(x_flat, seg, watt)

    out = pl.pallas_call(
        _dense_kernel,
        out_shape=jax.ShapeDtypeStruct((N_NODES, D), jnp.float32),
    )(acc, W1, W2, gamma.reshape(1, D), beta.reshape(1, D))
    return out
